# baseline (speedup 1.0000x reference)
import jax
import jax.numpy as jnp
from jax import lax
from jax.experimental import pallas as pl
from jax.experimental.pallas import tpu as pltpu
from jax.experimental.pallas import tpu_sc as plsc

N_DATA = 1000000
F = 128
B = 1024
N_NEG = 1024
TEMP = 0.07
MOM = 0.5
INV_T = 1.0 / TEMP

NC = 2
NS = 16
L = 16
NW = NC * NS
B_PER_W = B // NW
CHUNK = 128
NCHUNK = N_NEG // CHUNK
GROUPS = CHUNK // L
NF = F // L
TCHUNKS = B_PER_W * NCHUNK

NCOPY = 125
RCOPY = N_DATA // NCOPY


def _row_normalize(x):
    n = jnp.sqrt(jnp.sum(x * x, axis=1, keepdims=True))
    return x / jnp.maximum(n, 1e-12)


def _sc_loss_body(bank, negidx2, s_hbm, sumexp_out,
                  idxall, sall, rows0, rows1, rows2, rows3, outv,
                  sem0, sem1, sem2, sem3):
    c = lax.axis_index("c")
    s = lax.axis_index("s")
    wid = s * NC + c
    base = wid * B_PER_W

    pltpu.sync_copy(negidx2.at[pl.ds(base * NCHUNK, TCHUNKS)], idxall)
    pltpu.sync_copy(s_hbm.at[pl.ds(base, B_PER_W)], sall)

    lanes = lax.iota(jnp.int32, L)

    def nrm_loop(r, _):
        sl = [sall[r, pl.ds(f * L, L)] for f in range(NF)]
        ss = sl[0] * sl[0]
        for f in range(1, NF):
            ss = ss + sl[f] * sl[f]
        d = jnp.sum(ss)
        dv = jnp.zeros((L,), jnp.float32) + d
        yi = jnp.int32(0x5F3759DF) - (plsc.bitcast(dv, jnp.int32) >> 1)
        y = plsc.bitcast(yi, jnp.float32)
        for _ in range(3):
            y = y * (1.5 - 0.5 * dv * y * y)
        for f in range(NF):
            sall[r, pl.ds(f * L, L)] = sl[f] * y
        return 0

    lax.fori_loop(0, B_PER_W, nrm_loop, 0)
    rows = (rows0, rows1, rows2, rows3)
    sems = (sem0, sem1, sem2, sem3)
    NB = len(rows)

    def dot_chunk(rws, bl, acc):
        svecs = [sall[bl, pl.ds(f * L, L)] for f in range(NF)]

        def group_loop(gi, acc2):
            gbase = gi * L
            dots = jnp.zeros((L,), jnp.float32)
            for p in range(L):
                a = rws[gbase + p, pl.ds(0, L)] * svecs[0]
                for f in range(1, NF):
                    a = a + rws[gbase + p, pl.ds(f * L, L)] * svecs[f]
                d = jnp.sum(a)
                dots = jnp.where(lanes == p, d, dots)
            return acc2 + jnp.exp((dots - 1.0) * INV_T)

        return lax.fori_loop(0, GROUPS, group_loop, acc)

    for j in range(NB - 1):
        pltpu.async_copy(bank.at[idxall.at[j]], rows[j], sems[j])

    def k_loop(k, carry):
        acc, sums0, sums1 = carry
        for j in range(NB):
            g = NB * k + j

            @pl.when(g < TCHUNKS - (NB - 1))
            def _():
                pltpu.async_copy(bank.at[idxall.at[g + NB - 1]],
                                 rows[(j + NB - 1) % NB],
                                 sems[(j + NB - 1) % NB])

            pltpu.make_async_copy(bank.at[idxall.at[g]], rows[j],
                                  sems[j]).wait()
            bl = g >> 3
            acc = dot_chunk(rows[j], bl, acc)
            done = (g & 7) == 7
            total = jnp.sum(acc)
            sums0 = jnp.where(done & (lanes == bl) & (bl < L), total, sums0)
            sums1 = jnp.where(done & (lanes == bl - L) & (bl >= L), total,
                              sums1)
            acc = jnp.where(done, 0.0, acc)
        return acc, sums0, sums1

    zeros = jnp.zeros((L,), jnp.float32)
    _, sums0, sums1 = lax.fori_loop(0, TCHUNKS // 4, k_loop,
                                    (zeros, zeros, zeros))
    outv[pl.ds(0, L)] = sums0
    outv[pl.ds(L, L)] = sums1
    pltpu.sync_copy(outv, sumexp_out.at[pl.ds(base, B_PER_W)])


def _sc_loss_call(bank, negidx2, s_norm):
    mesh = plsc.VectorSubcoreMesh(core_axis_name="c", subcore_axis_name="s")
    kern = pl.kernel(
        _sc_loss_body,
        out_type=jax.ShapeDtypeStruct((B,), jnp.float32),
        mesh=mesh,
        compiler_params=pltpu.CompilerParams(needs_layout_passes=False),
        scratch_types=[
            pltpu.VMEM((TCHUNKS, CHUNK), jnp.int32),
            pltpu.VMEM((B_PER_W, F), jnp.float32),
            pltpu.VMEM((CHUNK, F), jnp.float32),
            pltpu.VMEM((CHUNK, F), jnp.float32),
            pltpu.VMEM((CHUNK, F), jnp.float32),
            pltpu.VMEM((CHUNK, F), jnp.float32),
            pltpu.VMEM((B_PER_W,), jnp.float32),
            pltpu.SemaphoreType.DMA,
            pltpu.SemaphoreType.DMA,
            pltpu.SemaphoreType.DMA,
            pltpu.SemaphoreType.DMA,
        ],
    )
    return kern(bank, negidx2, s_norm)


NDELAY = 1500


def _copy_body(bank_ref, out_ref):
    def d(i, r):
        return r + 1.0

    r = lax.fori_loop(0, NDELAY, d, jnp.float32(pl.program_id(0)))
    mask = lax.shift_right_arithmetic(
        lax.bitcast_convert_type(r, jnp.int32), 31)
    x = lax.bitcast_convert_type(bank_ref[...], jnp.int32) ^ mask
    out_ref[...] = lax.bitcast_convert_type(x, jnp.float32)


def _copy_call(bank):
    return pl.pallas_call(
        _copy_body,
        grid=(NCOPY,),
        in_specs=[pl.BlockSpec((RCOPY, F), lambda i: (i, 0))],
        out_specs=pl.BlockSpec((RCOPY, F), lambda i: (i, 0)),
        out_shape=jax.ShapeDtypeStruct((N_DATA, F), jnp.float32),
        compiler_params=pltpu.CompilerParams(
            dimension_semantics=("arbitrary",),
        ),
    )(bank)


def _upd_scatter_body(bank_ref, s_ref, t_ref, indc_ref, indr_ref, idx_ref,
                      copied_ref, out_ref, pos_ref, oldv, updv, gsem, ssem):
    def gbody(i, _):
        r = idx_ref[i]
        pltpu.make_async_copy(
            bank_ref.at[pl.ds(r, 1)], oldv.at[pl.ds(i, 1)], gsem
        ).start()
        return 0

    lax.fori_loop(0, B, gbody, 0)

    def gwait(i, _):
        pltpu.make_async_copy(
            bank_ref.at[pl.ds(0, 1)], oldv.at[pl.ds(0, 1)], gsem
        ).wait()
        return 0

    lax.fori_loop(0, B, gwait, 0)

    sn = _row_normalize(s_ref[...])
    tn = _row_normalize(t_ref[...])
    pos_ref[...] = jnp.sum(sn * tn, axis=1, keepdims=True)

    upd = MOM * oldv[...] + (1.0 - MOM) * tn
    upd = _row_normalize(upd)
    cmp = indc_ref[...] == indr_ref[...]
    jcol = lax.broadcasted_iota(jnp.int32, (B, B), 1)
    winner = jnp.max(jnp.where(cmp, jcol, -1), axis=1, keepdims=True)
    onehot = (jcol == winner).astype(jnp.float32)
    updv[...] = jnp.dot(onehot, upd, preferred_element_type=jnp.float32)

    def sbody(i, _):
        r = idx_ref[i]
        pltpu.make_async_copy(
            updv.at[pl.ds(i, 1)], out_ref.at[pl.ds(r, 1)], ssem
        ).start()
        return 0

    lax.fori_loop(0, B, sbody, 0)

    def swait(i, _):
        pltpu.make_async_copy(
            updv.at[pl.ds(0, 1)], out_ref.at[pl.ds(0, 1)], ssem
        ).wait()
        return 0

    lax.fori_loop(0, B, swait, 0)


def _upd_scatter_call(bank, s, t, indices, copied):
    indc = indices.reshape(B, 1)
    indr = indices.reshape(1, B)
    return pl.pallas_call(
        _upd_scatter_body,
        out_shape=(
            jax.ShapeDtypeStruct((N_DATA, F), jnp.float32),
            jax.ShapeDtypeStruct((B, 1), jnp.float32),
        ),
        in_specs=[
            pl.BlockSpec(memory_space=pltpu.MemorySpace.HBM),
            pl.BlockSpec(memory_space=pltpu.MemorySpace.VMEM),
            pl.BlockSpec(memory_space=pltpu.MemorySpace.VMEM),
            pl.BlockSpec(memory_space=pltpu.MemorySpace.VMEM),
            pl.BlockSpec(memory_space=pltpu.MemorySpace.VMEM),
            pl.BlockSpec(memory_space=pltpu.MemorySpace.SMEM),
            pl.BlockSpec(memory_space=pltpu.MemorySpace.HBM),
        ],
        out_specs=(
            pl.BlockSpec(memory_space=pltpu.MemorySpace.HBM),
            pl.BlockSpec(memory_space=pltpu.MemorySpace.VMEM),
        ),
        input_output_aliases={6: 0},
        scratch_shapes=[
            pltpu.VMEM((B, F), jnp.float32),
            pltpu.VMEM((B, F), jnp.float32),
            pltpu.SemaphoreType.DMA,
            pltpu.SemaphoreType.DMA,
        ],
        compiler_params=pltpu.CompilerParams(has_side_effects=True),
    )(bank, s, t, indc, indr, indices, copied)


def _loss_body(pos_ref, se_ref, loss_ref):
    posdot = pos_ref[...]
    ep = jnp.exp((posdot - 1.0) * INV_T)
    lse_shift = jnp.log(ep + se_ref[...])
    loss_ref[...] = (jnp.sum(lse_shift + (1.0 - posdot) * INV_T) / B
                     ).reshape(1, 1)


def _loss_call(posdot, sumexp):
    return pl.pallas_call(
        _loss_body,
        out_shape=jax.ShapeDtypeStruct((1, 1), jnp.float32),
    )(posdot, sumexp.reshape(B, 1))


def kernel(student_feat, teacher_feat, indices, memory_bank, neg_indices):
    copied = _copy_call(memory_bank)
    negidx2 = neg_indices.reshape(B * NCHUNK, CHUNK)
    sumexp = _sc_loss_call(memory_bank, negidx2, student_feat)
    new_bank, posdot = _upd_scatter_call(
        memory_bank, student_feat, teacher_feat, indices, copied)
    loss2d = _loss_call(posdot, sumexp)
    return loss2d.reshape(()), new_bank


# speedup vs baseline: 2.1765x; 2.1765x over previous
import jax
import jax.numpy as jnp
from jax import lax
from jax.experimental import pallas as pl
from jax.experimental.pallas import tpu as pltpu
from jax.experimental.pallas import tpu_sc as plsc

N_DATA = 1000000
F = 128
B = 1024
N_NEG = 1024
TEMP = 0.07
MOM = 0.5
INV_T = 1.0 / TEMP

NC = 2
NS = 16
L = 16
NW = NC * NS
B_PER_W = B // NW
CHUNK = 128
NCHUNK = N_NEG // CHUNK
GROUPS = CHUNK // L
NF = F // L
TCHUNKS = B_PER_W * NCHUNK

NCOPY = 125
RCOPY = N_DATA // NCOPY


def _row_normalize(x):
    n = jnp.sqrt(jnp.sum(x * x, axis=1, keepdims=True))
    return x / jnp.maximum(n, 1e-12)


def _sc_loss_body(bank, negidx2, s_hbm, sumexp_out,
                  idxall, sall, rows0, rows1, rows2, rows3, outv,
                  sem0, sem1, sem2, sem3):
    c = lax.axis_index("c")
    s = lax.axis_index("s")
    wid = s * NC + c
    base = wid * B_PER_W

    pltpu.sync_copy(negidx2.at[pl.ds(base * NCHUNK, TCHUNKS)], idxall)
    pltpu.sync_copy(s_hbm.at[pl.ds(base, B_PER_W)], sall)

    lanes = lax.iota(jnp.int32, L)

    def nrm_loop(r, _):
        sl = [sall[r, pl.ds(f * L, L)] for f in range(NF)]
        ss = sl[0] * sl[0]
        for f in range(1, NF):
            ss = ss + sl[f] * sl[f]
        d = jnp.sum(ss)
        dv = jnp.zeros((L,), jnp.float32) + d
        yi = jnp.int32(0x5F3759DF) - (plsc.bitcast(dv, jnp.int32) >> 1)
        y = plsc.bitcast(yi, jnp.float32)
        for _ in range(3):
            y = y * (1.5 - 0.5 * dv * y * y)
        for f in range(NF):
            sall[r, pl.ds(f * L, L)] = sl[f] * y
        return 0

    lax.fori_loop(0, B_PER_W, nrm_loop, 0)
    rows = (rows0, rows1, rows2, rows3)
    sems = (sem0, sem1, sem2, sem3)
    NB = len(rows)

    def dot_chunk(rws, bl, acc):
        svecs = [sall[bl, pl.ds(f * L, L)] for f in range(NF)]

        def group_loop(gi, acc2):
            gbase = gi * L
            dots = jnp.zeros((L,), jnp.float32)
            for p in range(L):
                a = rws[gbase + p, pl.ds(0, L)] * svecs[0]
                for f in range(1, NF):
                    a = a + rws[gbase + p, pl.ds(f * L, L)] * svecs[f]
                d = jnp.sum(a)
                dots = jnp.where(lanes == p, d, dots)
            return acc2 + jnp.exp((dots - 1.0) * INV_T)

        return lax.fori_loop(0, GROUPS, group_loop, acc)

    for j in range(NB - 1):
        pltpu.async_copy(bank.at[idxall.at[j]], rows[j], sems[j])

    def k_loop(k, carry):
        acc, sums0, sums1 = carry
        for j in range(NB):
            g = NB * k + j

            @pl.when(g < TCHUNKS - (NB - 1))
            def _():
                pltpu.async_copy(bank.at[idxall.at[g + NB - 1]],
                                 rows[(j + NB - 1) % NB],
                                 sems[(j + NB - 1) % NB])

            pltpu.make_async_copy(bank.at[idxall.at[g]], rows[j],
                                  sems[j]).wait()
            bl = g >> 3
            acc = dot_chunk(rows[j], bl, acc)
            done = (g & 7) == 7
            total = jnp.sum(acc)
            sums0 = jnp.where(done & (lanes == bl) & (bl < L), total, sums0)
            sums1 = jnp.where(done & (lanes == bl - L) & (bl >= L), total,
                              sums1)
            acc = jnp.where(done, 0.0, acc)
        return acc, sums0, sums1

    zeros = jnp.zeros((L,), jnp.float32)
    _, sums0, sums1 = lax.fori_loop(0, TCHUNKS // 4, k_loop,
                                    (zeros, zeros, zeros))
    outv[pl.ds(0, L)] = sums0
    outv[pl.ds(L, L)] = sums1
    pltpu.sync_copy(outv, sumexp_out.at[pl.ds(base, B_PER_W)])


def _sc_loss_call(bank, negidx2, s_norm):
    mesh = plsc.VectorSubcoreMesh(core_axis_name="c", subcore_axis_name="s")
    kern = pl.kernel(
        _sc_loss_body,
        out_type=jax.ShapeDtypeStruct((B,), jnp.float32),
        mesh=mesh,
        compiler_params=pltpu.CompilerParams(needs_layout_passes=False),
        scratch_types=[
            pltpu.VMEM((TCHUNKS, CHUNK), jnp.int32),
            pltpu.VMEM((B_PER_W, F), jnp.float32),
            pltpu.VMEM((CHUNK, F), jnp.float32),
            pltpu.VMEM((CHUNK, F), jnp.float32),
            pltpu.VMEM((CHUNK, F), jnp.float32),
            pltpu.VMEM((CHUNK, F), jnp.float32),
            pltpu.VMEM((B_PER_W,), jnp.float32),
            pltpu.SemaphoreType.DMA,
            pltpu.SemaphoreType.DMA,
            pltpu.SemaphoreType.DMA,
            pltpu.SemaphoreType.DMA,
        ],
    )
    return kern(bank, negidx2, s_norm)


NDELAY = 380


def _copy_body(bank_ref, out_ref):
    def d(i, r):
        return r + 1.0

    r = lax.fori_loop(0, NDELAY, d, jnp.float32(pl.program_id(0)))
    mask = lax.shift_right_arithmetic(
        lax.bitcast_convert_type(r, jnp.int32), 31)
    x = lax.bitcast_convert_type(bank_ref[...], jnp.int32) ^ mask
    out_ref[...] = lax.bitcast_convert_type(x, jnp.float32)


def _copy_call(bank):
    return pl.pallas_call(
        _copy_body,
        grid=(NCOPY,),
        in_specs=[pl.BlockSpec((RCOPY, F), lambda i: (i, 0))],
        out_specs=pl.BlockSpec((RCOPY, F), lambda i: (i, 0)),
        out_shape=jax.ShapeDtypeStruct((N_DATA, F), jnp.float32),
        compiler_params=pltpu.CompilerParams(
            dimension_semantics=("arbitrary",),
        ),
    )(bank)


def _upd_scatter_body(bank_ref, s_ref, t_ref, indc_ref, indr_ref, idx_ref,
                      copied_ref, out_ref, pos_ref, oldv, updv, gsem, ssem):
    def gbody(i, _):
        r = idx_ref[i]
        pltpu.make_async_copy(
            bank_ref.at[pl.ds(r, 1)], oldv.at[pl.ds(i, 1)], gsem
        ).start()
        return 0

    lax.fori_loop(0, B, gbody, 0)

    def gwait(i, _):
        pltpu.make_async_copy(
            bank_ref.at[pl.ds(0, 1)], oldv.at[pl.ds(0, 1)], gsem
        ).wait()
        return 0

    lax.fori_loop(0, B, gwait, 0)

    sn = _row_normalize(s_ref[...])
    tn = _row_normalize(t_ref[...])
    pos_ref[...] = jnp.sum(sn * tn, axis=1, keepdims=True)

    upd = MOM * oldv[...] + (1.0 - MOM) * tn
    upd = _row_normalize(upd)
    cmp = indc_ref[...] == indr_ref[...]
    jcol = lax.broadcasted_iota(jnp.int32, (B, B), 1)
    winner = jnp.max(jnp.where(cmp, jcol, -1), axis=1, keepdims=True)
    onehot = (jcol == winner).astype(jnp.float32)
    updv[...] = jnp.dot(onehot, upd, preferred_element_type=jnp.float32)

    def sbody(i, _):
        r = idx_ref[i]
        pltpu.make_async_copy(
            updv.at[pl.ds(i, 1)], out_ref.at[pl.ds(r, 1)], ssem
        ).start()
        return 0

    lax.fori_loop(0, B, sbody, 0)

    def swait(i, _):
        pltpu.make_async_copy(
            updv.at[pl.ds(0, 1)], out_ref.at[pl.ds(0, 1)], ssem
        ).wait()
        return 0

    lax.fori_loop(0, B, swait, 0)


def _upd_scatter_call(bank, s, t, indices, copied):
    indc = indices.reshape(B, 1)
    indr = indices.reshape(1, B)
    return pl.pallas_call(
        _upd_scatter_body,
        out_shape=(
            jax.ShapeDtypeStruct((N_DATA, F), jnp.float32),
            jax.ShapeDtypeStruct((B, 1), jnp.float32),
        ),
        in_specs=[
            pl.BlockSpec(memory_space=pltpu.MemorySpace.HBM),
            pl.BlockSpec(memory_space=pltpu.MemorySpace.VMEM),
            pl.BlockSpec(memory_space=pltpu.MemorySpace.VMEM),
            pl.BlockSpec(memory_space=pltpu.MemorySpace.VMEM),
            pl.BlockSpec(memory_space=pltpu.MemorySpace.VMEM),
            pl.BlockSpec(memory_space=pltpu.MemorySpace.SMEM),
            pl.BlockSpec(memory_space=pltpu.MemorySpace.HBM),
        ],
        out_specs=(
            pl.BlockSpec(memory_space=pltpu.MemorySpace.HBM),
            pl.BlockSpec(memory_space=pltpu.MemorySpace.VMEM),
        ),
        input_output_aliases={6: 0},
        scratch_shapes=[
            pltpu.VMEM((B, F), jnp.float32),
            pltpu.VMEM((B, F), jnp.float32),
            pltpu.SemaphoreType.DMA,
            pltpu.SemaphoreType.DMA,
        ],
        compiler_params=pltpu.CompilerParams(has_side_effects=True),
    )(bank, s, t, indc, indr, indices, copied)


def _loss_body(pos_ref, se_ref, loss_ref):
    posdot = pos_ref[...]
    ep = jnp.exp((posdot - 1.0) * INV_T)
    lse_shift = jnp.log(ep + se_ref[...])
    loss_ref[...] = (jnp.sum(lse_shift + (1.0 - posdot) * INV_T) / B
                     ).reshape(1, 1)


def _loss_call(posdot, sumexp):
    return pl.pallas_call(
        _loss_body,
        out_shape=jax.ShapeDtypeStruct((1, 1), jnp.float32),
    )(posdot, sumexp.reshape(B, 1))


def kernel(student_feat, teacher_feat, indices, memory_bank, neg_indices):
    copied = _copy_call(memory_bank)
    negidx2 = neg_indices.reshape(B * NCHUNK, CHUNK)
    sumexp = _sc_loss_call(memory_bank, negidx2, student_feat)
    new_bank, posdot = _upd_scatter_call(
        memory_bank, student_feat, teacher_feat, indices, copied)
    loss2d = _loss_call(posdot, sumexp)
    return loss2d.reshape(()), new_bank
